# Initial kernel scaffold; baseline (speedup 1.0000x reference)
import jax
import jax.numpy as jnp
from jax.experimental import pallas as pl


def kernel(input, values):
    raise NotImplementedError("write your pallas kernel here")


# speedup vs baseline: 4.1012x; 4.1012x over previous
import functools

import jax
import jax.numpy as jnp
from jax import lax
from jax.experimental import pallas as pl
from jax.experimental.pallas import tpu as pltpu
from jax.experimental.pallas import tpu_sc as plsc

P = 256
NPIX = P * P
VPAD = 1024


def _gather_bias_sc(values_padded):
    info = plsc.get_sparse_core_info()
    nc, ns, L = info.num_cores, info.num_subcores, info.num_lanes
    nw = nc * ns
    per_w = NPIX // nw
    n_vec = per_w // L

    @functools.partial(
        pl.kernel,
        mesh=plsc.VectorSubcoreMesh(core_axis_name="c", subcore_axis_name="s"),
        out_type=jax.ShapeDtypeStruct((NPIX,), jnp.float32),
        scratch_types=[
            pltpu.VMEM((VPAD,), jnp.float32),
            pltpu.VMEM((per_w,), jnp.float32),
        ],
        compiler_params=pltpu.CompilerParams(needs_layout_passes=False),
    )
    def gather_kernel(vals_hbm, out_hbm, vals_v, out_v):
        wid = lax.axis_index("s") * nc + lax.axis_index("c")
        base = wid * per_w
        pltpu.sync_copy(vals_hbm, vals_v)
        lane = lax.iota(jnp.int32, L)

        def body(i, carry):
            p = base + i * L + lane
            a = lax.shift_right_logical(p, 8)
            b = jnp.bitwise_and(p, 255)
            idx = (
                jnp.bitwise_and(a, 15) - jnp.bitwise_and(b, 15) + 15
            ) + 31 * (
                lax.shift_right_logical(a, 4) - lax.shift_right_logical(b, 4) + 15
            )
            out_v[pl.ds(i * L, L)] = plsc.load_gather(vals_v, [idx])
            return carry

        lax.fori_loop(0, n_vec, body, 0)
        pltpu.sync_copy(out_v, out_hbm.at[pl.ds(base, per_w)])

    return gather_kernel(values_padded)


def _add_body(x_ref, b_ref, o_ref):
    o_ref[...] = x_ref[...] + b_ref[...]


def _add_tc(x3, bias2):
    rows = 8
    return pl.pallas_call(
        _add_body,
        grid=(x3.shape[0] // rows,),
        in_specs=[
            pl.BlockSpec((rows, P, P), lambda i: (i, 0, 0)),
            pl.BlockSpec((P, P), lambda i: (0, 0)),
        ],
        out_specs=pl.BlockSpec((rows, P, P), lambda i: (i, 0, 0)),
        out_shape=jax.ShapeDtypeStruct(x3.shape, jnp.float32),
        compiler_params=pltpu.CompilerParams(
            dimension_semantics=("arbitrary",),
        ),
    )(x3, bias2)


def kernel(input, values):
    vals_padded = jnp.pad(values, (0, VPAD - values.shape[0]))
    bias = _gather_bias_sc(vals_padded).reshape(P, P)
    x3 = input.reshape(-1, P, P)
    out = _add_tc(x3, bias)
    return out.reshape(input.shape)


# speedup vs baseline: 4.4834x; 1.0932x over previous
import functools

import jax
import jax.numpy as jnp
from jax import lax
from jax.experimental import pallas as pl
from jax.experimental.pallas import tpu as pltpu
from jax.experimental.pallas import tpu_sc as plsc

P = 256
NPIX = P * P
VPAD = 1024


def _gather_bias_sc(values_padded):
    info = plsc.get_sparse_core_info()
    nc, ns, L = info.num_cores, info.num_subcores, info.num_lanes
    nw = nc * ns
    per_w = NPIX // nw
    n_vec = per_w // L

    @functools.partial(
        pl.kernel,
        mesh=plsc.VectorSubcoreMesh(core_axis_name="c", subcore_axis_name="s"),
        out_type=jax.ShapeDtypeStruct((NPIX,), jnp.float32),
        scratch_types=[
            pltpu.VMEM((VPAD,), jnp.float32),
            pltpu.VMEM((per_w,), jnp.float32),
        ],
        compiler_params=pltpu.CompilerParams(needs_layout_passes=False),
    )
    def gather_kernel(vals_hbm, out_hbm, vals_v, out_v):
        wid = lax.axis_index("s") * nc + lax.axis_index("c")
        base = wid * per_w
        pltpu.sync_copy(vals_hbm, vals_v)
        lane = lax.iota(jnp.int32, L)

        def body(i, carry):
            p = base + i * L + lane
            a = lax.shift_right_logical(p, 8)
            b = jnp.bitwise_and(p, 255)
            idx = (
                jnp.bitwise_and(a, 15) - jnp.bitwise_and(b, 15) + 15
            ) + 31 * (
                lax.shift_right_logical(a, 4) - lax.shift_right_logical(b, 4) + 15
            )
            out_v[pl.ds(i * L, L)] = plsc.load_gather(vals_v, [idx])
            return carry

        lax.fori_loop(0, n_vec, body, 0)
        pltpu.sync_copy(out_v, out_hbm.at[pl.ds(base, per_w)])

    return gather_kernel(values_padded)


def _add_body(x_ref, b_ref, o_ref):
    o_ref[...] = x_ref[...] + b_ref[...]


def _add_tc(x3, bias2):
    rows = 32
    return pl.pallas_call(
        _add_body,
        grid=(x3.shape[0] // rows,),
        in_specs=[
            pl.BlockSpec((rows, P, P), lambda i: (i, 0, 0)),
            pl.BlockSpec((P, P), lambda i: (0, 0)),
        ],
        out_specs=pl.BlockSpec((rows, P, P), lambda i: (i, 0, 0)),
        out_shape=jax.ShapeDtypeStruct(x3.shape, jnp.float32),
        compiler_params=pltpu.CompilerParams(
            dimension_semantics=("arbitrary",),
        ),
    )(x3, bias2)


def kernel(input, values):
    vals_padded = jnp.pad(values, (0, VPAD - values.shape[0]))
    bias = _gather_bias_sc(vals_padded).reshape(P, P)
    x3 = input.reshape(-1, P, P)
    out = _add_tc(x3, bias)
    return out.reshape(input.shape)
